# baseline (speedup 1.0000x reference)
import jax
import jax.numpy as jnp
from jax import lax
from jax.experimental import pallas as pl
from jax.experimental.pallas import tpu as pltpu
from jax.experimental.pallas import tpu_sc as plsc

N = 100000
H = 128
G = 32
NB = G * G
NB1 = NB + 1
R = 2000
NBLK = N // R

CHUNK = 128
NFULL = N // CHUNK
TAIL = N - NFULL * CHUNK
NCHUNK = NFULL + 1
NW = 32
ITERS = (NCHUNK + NW - 1) // NW

_mesh = plsc.VectorSubcoreMesh(core_axis_name="c", subcore_axis_name="s")


def _sigmoid(x):
    return 1.0 / (1.0 + jnp.exp(-x))


def _grid_id(coords):
    x = jnp.clip(coords[:, 0], 0.0, 1.0)
    y = jnp.clip(coords[:, 1], 0.0, 1.0)
    ix = jnp.clip(jnp.floor(x * G).astype(jnp.int32), 0, G - 1)
    iy = jnp.clip(jnp.floor(y * G).astype(jnp.int32), 0, G - 1)
    return ix * G + iy


def _lstm_kernel(coords_ref, hid_ref, cell_ref, wih_ref, whh_ref, b_ref,
                 cnew_ref, hnew_ref, gid_ref):
    coords = coords_ref[...]
    x = coords[:, 0:1]
    y = coords[:, 1:2]
    gates = (jnp.dot(hid_ref[...], whh_ref[...], preferred_element_type=jnp.float32)
             + x * wih_ref[0:1, :] + y * wih_ref[1:2, :] + b_ref[...])
    ii = _sigmoid(gates[:, :H])
    ff = _sigmoid(gates[:, H:2 * H])
    gg = jnp.tanh(gates[:, 2 * H:3 * H])
    oo = _sigmoid(gates[:, 3 * H:])
    c_new = ff * cell_ref[...] + ii * gg
    cnew_ref[...] = c_new
    hnew_ref[...] = oo * jnp.tanh(c_new)
    gid_ref[0, 0, :] = _grid_id(coords)


def _sc_scatter_kernel(hnew_hbm, gid_hbm, zeros_hbm, pbins_hbm,
                       rows_v, idx_v, bins_sh):
    c = lax.axis_index("c")
    s = lax.axis_index("s")
    wid = c * 16 + s
    pltpu.sync_copy(zeros_hbm, bins_sh.at[pl.ds(s * 64, 64), :])
    plsc.subcore_barrier()

    @pl.loop(0, ITERS)
    def _(it):
        cid = it * NW + wid

        @pl.when(cid < NFULL)
        def _():
            pltpu.sync_copy(hnew_hbm.at[pl.ds(cid * CHUNK, CHUNK), :], rows_v)

        @pl.when(cid == NFULL)
        def _():
            pltpu.sync_copy(hnew_hbm.at[pl.ds(NFULL * CHUNK, TAIL), :],
                            rows_v.at[pl.ds(0, TAIL), :])

        @pl.when(cid < NCHUNK)
        def _():
            pltpu.sync_copy(gid_hbm.at[pl.ds(cid, 1), :], idx_v)
            pltpu.sync_copy(rows_v, bins_sh.at[idx_v.at[0]], add=True)

    plsc.subcore_barrier()
    pltpu.sync_copy(bins_sh.at[pl.ds(s * 64, 64), :],
                    pbins_hbm.at[c, pl.ds(s * 64, 64), :])


def _sc_merge_kernel(pbins_hbm, bins_hbm, a_v, b_v):
    c = lax.axis_index("c")
    s = lax.axis_index("s")
    wid = c * 16 + s
    base = wid * (NB // NW)
    pltpu.sync_copy(pbins_hbm.at[0, pl.ds(base, NB // NW), :], a_v)
    pltpu.sync_copy(pbins_hbm.at[1, pl.ds(base, NB // NW), :], b_v)

    @pl.loop(0, NB // NW)
    def _(i):
        for j in range(H // 16):
            sl = (i, pl.ds(j * 16, 16))
            a_v[sl] = a_v[sl] + b_v[sl]

    pltpu.sync_copy(a_v, bins_hbm.at[pl.ds(base, NB // NW), :])


def _sc_gather_kernel(bins_hbm, gid_hbm, out_hbm, rows_v, idx_v):
    c = lax.axis_index("c")
    s = lax.axis_index("s")
    wid = c * 16 + s

    @pl.loop(0, ITERS)
    def _(it):
        cid = it * NW + wid

        @pl.when(cid < NCHUNK)
        def _():
            pltpu.sync_copy(gid_hbm.at[pl.ds(cid, 1), :], idx_v)
            pltpu.sync_copy(bins_hbm.at[idx_v.at[0]], rows_v)

        @pl.when(cid < NFULL)
        def _():
            pltpu.sync_copy(rows_v, out_hbm.at[pl.ds(cid * CHUNK, CHUNK), :])

        @pl.when(cid == NFULL)
        def _():
            pltpu.sync_copy(rows_v.at[pl.ds(0, TAIL), :],
                            out_hbm.at[pl.ds(NFULL * CHUNK, TAIL), :])


def kernel(coords, hidden_state, cell_state, W_ih, W_hh, b_ih, b_hh):
    wih = W_ih.T
    whh = W_hh.T
    b = (b_ih + b_hh)[None, :]

    c_new, h_new, gid3 = pl.pallas_call(
        _lstm_kernel,
        grid=(NBLK,),
        in_specs=[
            pl.BlockSpec((R, 2), lambda i: (i, 0)),
            pl.BlockSpec((R, H), lambda i: (i, 0)),
            pl.BlockSpec((R, H), lambda i: (i, 0)),
            pl.BlockSpec((2, 4 * H), lambda i: (0, 0)),
            pl.BlockSpec((H, 4 * H), lambda i: (0, 0)),
            pl.BlockSpec((1, 4 * H), lambda i: (0, 0)),
        ],
        out_specs=[
            pl.BlockSpec((R, H), lambda i: (i, 0)),
            pl.BlockSpec((R, H), lambda i: (i, 0)),
            pl.BlockSpec((1, 1, R), lambda i: (i, 0, 0)),
        ],
        out_shape=[
            jax.ShapeDtypeStruct((N, H), jnp.float32),
            jax.ShapeDtypeStruct((N, H), jnp.float32),
            jax.ShapeDtypeStruct((NBLK, 1, R), jnp.int32),
        ],
    )(coords, hidden_state, cell_state, wih, whh, b)

    gid2 = jnp.pad(gid3.reshape(N), (0, NCHUNK * CHUNK - N),
                   constant_values=NB).reshape(NCHUNK, CHUNK)
    zeros = jnp.zeros((64, H), jnp.float32)

    scatter = pl.kernel(
        _sc_scatter_kernel,
        out_type=jax.ShapeDtypeStruct((2, NB, H), jnp.float32),
        mesh=_mesh,
        scratch_types=[
            pltpu.VMEM((CHUNK, H), jnp.float32),
            pltpu.VMEM((1, CHUNK), jnp.int32),
            pltpu.VMEM_SHARED((NB1, H), jnp.float32),
        ],
    )
    pbins = scatter(h_new, gid2, zeros)

    merge = pl.kernel(
        _sc_merge_kernel,
        out_type=jax.ShapeDtypeStruct((NB1, H), jnp.float32),
        mesh=_mesh,
        scratch_types=[
            pltpu.VMEM((NB // NW, H), jnp.float32),
            pltpu.VMEM((NB // NW, H), jnp.float32),
        ],
    )
    bins = merge(pbins)

    gather = pl.kernel(
        _sc_gather_kernel,
        out_type=jax.ShapeDtypeStruct((N, H), jnp.float32),
        mesh=_mesh,
        scratch_types=[
            pltpu.VMEM((CHUNK, H), jnp.float32),
            pltpu.VMEM((1, CHUNK), jnp.int32),
        ],
    )
    h_social = gather(bins, gid2)

    return (h_social, c_new)


# speedup vs baseline: 1.0018x; 1.0018x over previous
import jax
import jax.numpy as jnp
from jax import lax
from jax.experimental import pallas as pl
from jax.experimental.pallas import tpu as pltpu
from jax.experimental.pallas import tpu_sc as plsc

N = 100000
H = 128
G = 32
NB = G * G
NB1 = NB + 1
R = 2000
NBLK = N // R

CHUNK = 128
NFULL = N // CHUNK
TAIL = N - NFULL * CHUNK
NCHUNK = NFULL + 1
NW = 32
ITERS = (NCHUNK + NW - 1) // NW

_mesh = plsc.VectorSubcoreMesh(core_axis_name="c", subcore_axis_name="s")


def _sigmoid(x):
    return 1.0 / (1.0 + jnp.exp(-x))


def _grid_id(coords):
    x = jnp.clip(coords[:, 0], 0.0, 1.0)
    y = jnp.clip(coords[:, 1], 0.0, 1.0)
    ix = jnp.clip(jnp.floor(x * G).astype(jnp.int32), 0, G - 1)
    iy = jnp.clip(jnp.floor(y * G).astype(jnp.int32), 0, G - 1)
    return ix * G + iy


def _tsig(x):
    return 0.5 * jnp.tanh(0.5 * x) + 0.5


def _lstm_kernel(coords_ref, hid_ref, cell_ref, feat_ref, whh_ref, wf_ref,
                 cnew_ref, hnew_ref, gid_ref):
    hid16 = hid_ref[...].astype(jnp.bfloat16)
    gates = (jnp.dot(hid16, whh_ref[...], preferred_element_type=jnp.float32)
             + jnp.dot(feat_ref[...], wf_ref[...], preferred_element_type=jnp.float32))
    ii = _tsig(gates[:, :H])
    ff = _tsig(gates[:, H:2 * H])
    gg = jnp.tanh(gates[:, 2 * H:3 * H])
    oo = _tsig(gates[:, 3 * H:])
    c_new = ff * cell_ref[...] + ii * gg
    cnew_ref[...] = c_new
    hnew_ref[...] = oo * jnp.tanh(c_new)
    gid_ref[0, 0, :] = _grid_id(coords_ref[...])


def _sc_scatter_kernel(hnew_hbm, gid_hbm, zeros_hbm, pbins_hbm,
                       rows_v, idx_v, bins_sh):
    c = lax.axis_index("c")
    s = lax.axis_index("s")
    wid = c * 16 + s
    pltpu.sync_copy(zeros_hbm, bins_sh.at[pl.ds(s * 64, 64), :])
    plsc.subcore_barrier()

    @pl.loop(0, ITERS)
    def _(it):
        cid = it * NW + wid

        @pl.when(cid < NFULL)
        def _():
            pltpu.sync_copy(hnew_hbm.at[pl.ds(cid * CHUNK, CHUNK), :], rows_v)

        @pl.when(cid == NFULL)
        def _():
            pltpu.sync_copy(hnew_hbm.at[pl.ds(NFULL * CHUNK, TAIL), :],
                            rows_v.at[pl.ds(0, TAIL), :])

        @pl.when(cid < NCHUNK)
        def _():
            pltpu.sync_copy(gid_hbm.at[pl.ds(cid, 1), :], idx_v)
            pltpu.sync_copy(rows_v, bins_sh.at[idx_v.at[0]], add=True)

    plsc.subcore_barrier()
    pltpu.sync_copy(bins_sh.at[pl.ds(s * 64, 64), :],
                    pbins_hbm.at[c, pl.ds(s * 64, 64), :])


def _sc_merge_kernel(pbins_hbm, bins_hbm, a_v, b_v):
    c = lax.axis_index("c")
    s = lax.axis_index("s")
    wid = c * 16 + s
    base = wid * (NB // NW)
    pltpu.sync_copy(pbins_hbm.at[0, pl.ds(base, NB // NW), :], a_v)
    pltpu.sync_copy(pbins_hbm.at[1, pl.ds(base, NB // NW), :], b_v)

    @pl.loop(0, NB // NW)
    def _(i):
        for j in range(H // 16):
            sl = (i, pl.ds(j * 16, 16))
            a_v[sl] = a_v[sl] + b_v[sl]

    pltpu.sync_copy(a_v, bins_hbm.at[pl.ds(base, NB // NW), :])


def _sc_gather_kernel(bins_hbm, gid_hbm, out_hbm, rows_v, idx_v):
    c = lax.axis_index("c")
    s = lax.axis_index("s")
    wid = c * 16 + s

    @pl.loop(0, ITERS)
    def _(it):
        cid = it * NW + wid

        @pl.when(cid < NCHUNK)
        def _():
            pltpu.sync_copy(gid_hbm.at[pl.ds(cid, 1), :], idx_v)
            pltpu.sync_copy(bins_hbm.at[idx_v.at[0]], rows_v)

        @pl.when(cid < NFULL)
        def _():
            pltpu.sync_copy(rows_v, out_hbm.at[pl.ds(cid * CHUNK, CHUNK), :])

        @pl.when(cid == NFULL)
        def _():
            pltpu.sync_copy(rows_v.at[pl.ds(0, TAIL), :],
                            out_hbm.at[pl.ds(NFULL * CHUNK, TAIL), :])


def kernel(coords, hidden_state, cell_state, W_ih, W_hh, b_ih, b_hh):
    whh = W_hh.T.astype(jnp.bfloat16)
    wf = jnp.concatenate(
        [W_ih.T, (b_ih + b_hh)[None, :], jnp.zeros((5, 4 * H), jnp.float32)],
        axis=0).astype(jnp.bfloat16)
    feat = jnp.concatenate(
        [coords, jnp.ones((N, 1), jnp.float32), jnp.zeros((N, 5), jnp.float32)],
        axis=1).astype(jnp.bfloat16)

    c_new, h_new, gid3 = pl.pallas_call(
        _lstm_kernel,
        grid=(NBLK,),
        in_specs=[
            pl.BlockSpec((R, 2), lambda i: (i, 0)),
            pl.BlockSpec((R, H), lambda i: (i, 0)),
            pl.BlockSpec((R, H), lambda i: (i, 0)),
            pl.BlockSpec((R, 8), lambda i: (i, 0)),
            pl.BlockSpec((H, 4 * H), lambda i: (0, 0)),
            pl.BlockSpec((8, 4 * H), lambda i: (0, 0)),
        ],
        out_specs=[
            pl.BlockSpec((R, H), lambda i: (i, 0)),
            pl.BlockSpec((R, H), lambda i: (i, 0)),
            pl.BlockSpec((1, 1, R), lambda i: (i, 0, 0)),
        ],
        out_shape=[
            jax.ShapeDtypeStruct((N, H), jnp.float32),
            jax.ShapeDtypeStruct((N, H), jnp.float32),
            jax.ShapeDtypeStruct((NBLK, 1, R), jnp.int32),
        ],
    )(coords, hidden_state, cell_state, feat, whh, wf)

    gid2 = jnp.pad(gid3.reshape(N), (0, NCHUNK * CHUNK - N),
                   constant_values=NB).reshape(NCHUNK, CHUNK)
    zeros = jnp.zeros((64, H), jnp.float32)

    scatter = pl.kernel(
        _sc_scatter_kernel,
        out_type=jax.ShapeDtypeStruct((2, NB, H), jnp.float32),
        mesh=_mesh,
        scratch_types=[
            pltpu.VMEM((CHUNK, H), jnp.float32),
            pltpu.VMEM((1, CHUNK), jnp.int32),
            pltpu.VMEM_SHARED((NB1, H), jnp.float32),
        ],
    )
    pbins = scatter(h_new, gid2, zeros)

    merge = pl.kernel(
        _sc_merge_kernel,
        out_type=jax.ShapeDtypeStruct((NB1, H), jnp.float32),
        mesh=_mesh,
        scratch_types=[
            pltpu.VMEM((NB // NW, H), jnp.float32),
            pltpu.VMEM((NB // NW, H), jnp.float32),
        ],
    )
    bins = merge(pbins)

    gather = pl.kernel(
        _sc_gather_kernel,
        out_type=jax.ShapeDtypeStruct((N, H), jnp.float32),
        mesh=_mesh,
        scratch_types=[
            pltpu.VMEM((CHUNK, H), jnp.float32),
            pltpu.VMEM((1, CHUNK), jnp.int32),
        ],
    )
    h_social = gather(bins, gid2)

    return (h_social, c_new)


# speedup vs baseline: 1.1168x; 1.1148x over previous
import jax
import jax.numpy as jnp
from jax import lax
from jax.experimental import pallas as pl
from jax.experimental.pallas import tpu as pltpu
from jax.experimental.pallas import tpu_sc as plsc

N = 100000
H = 128
G = 32
NB = G * G
NB1 = NB + 1
R = 2000
NBLK = N // R

CHUNK = 128
NFULL = N // CHUNK
TAIL = N - NFULL * CHUNK
NCHUNK = NFULL + 1
NW = 32
ITERS = (NCHUNK + NW - 1) // NW

_mesh = plsc.VectorSubcoreMesh(core_axis_name="c", subcore_axis_name="s")


def _tsig(x):
    return 0.5 * jnp.tanh(0.5 * x) + 0.5


def _lstm_kernel(hid_ref, cell_ref, feat_ref, whh_ref, wf_ref,
                 cnew_ref, hnew_ref):
    hid16 = hid_ref[...].astype(jnp.bfloat16)
    feat_blk = feat_ref[0]
    gates = (jnp.dot(hid16, whh_ref[...], preferred_element_type=jnp.float32)
             + lax.dot_general(feat_blk, wf_ref[...],
                               (((0,), (0,)), ((), ())),
                               preferred_element_type=jnp.float32))
    ii = _tsig(gates[:, :H])
    ff = _tsig(gates[:, H:2 * H])
    gg = jnp.tanh(gates[:, 2 * H:3 * H])
    oo = _tsig(gates[:, 3 * H:])
    c_new = ff * cell_ref[...] + ii * gg
    cnew_ref[...] = c_new
    hnew_ref[...] = oo * jnp.tanh(c_new)


def _sc_ids(xs_hbm, ys_hbm, x_v, y_v, idx_v, cid):
    @pl.when(cid < NFULL)
    def _():
        pltpu.sync_copy(xs_hbm.at[pl.ds(cid * CHUNK, CHUNK)], x_v)
        pltpu.sync_copy(ys_hbm.at[pl.ds(cid * CHUNK, CHUNK)], y_v)

    @pl.when(cid == NFULL)
    def _():
        pltpu.sync_copy(xs_hbm.at[pl.ds(NFULL * CHUNK, TAIL)],
                        x_v.at[pl.ds(0, TAIL)])
        pltpu.sync_copy(ys_hbm.at[pl.ds(NFULL * CHUNK, TAIL)],
                        y_v.at[pl.ds(0, TAIL)])

    for j in range(CHUNK // 16):
        xs = x_v[pl.ds(16 * j, 16)]
        ys = y_v[pl.ds(16 * j, 16)]
        ix = jnp.clip((jnp.clip(xs, 0.0, 1.0) * G).astype(jnp.int32), 0, G - 1)
        iy = jnp.clip((jnp.clip(ys, 0.0, 1.0) * G).astype(jnp.int32), 0, G - 1)
        ids = ix * G + iy
        row = cid * CHUNK + 16 * j + lax.iota(jnp.int32, 16)
        idx_v[0, pl.ds(16 * j, 16)] = jnp.where(row < N, ids, NB)


def _sc_scatter_kernel(hnew_hbm, xs_hbm, ys_hbm, zeros_hbm, pbins_hbm,
                       rows_v, idx_v, x_v, y_v, bins_sh):
    c = lax.axis_index("c")
    s = lax.axis_index("s")
    wid = c * 16 + s
    pltpu.sync_copy(zeros_hbm, bins_sh.at[pl.ds(s * 64, 64), :])
    plsc.subcore_barrier()

    @pl.loop(0, ITERS)
    def _(it):
        cid = it * NW + wid

        @pl.when(cid < NFULL)
        def _():
            pltpu.sync_copy(hnew_hbm.at[pl.ds(cid * CHUNK, CHUNK), :], rows_v)

        @pl.when(cid == NFULL)
        def _():
            pltpu.sync_copy(hnew_hbm.at[pl.ds(NFULL * CHUNK, TAIL), :],
                            rows_v.at[pl.ds(0, TAIL), :])

        @pl.when(cid < NCHUNK)
        def _():
            _sc_ids(xs_hbm, ys_hbm, x_v, y_v, idx_v, cid)
            pltpu.sync_copy(rows_v, bins_sh.at[idx_v.at[0]], add=True)

    plsc.subcore_barrier()
    pltpu.sync_copy(bins_sh.at[pl.ds(s * 64, 64), :],
                    pbins_hbm.at[c, pl.ds(s * 64, 64), :])


def _sc_merge_kernel(pbins_hbm, bins_hbm, a_v, b_v):
    c = lax.axis_index("c")
    s = lax.axis_index("s")
    wid = c * 16 + s
    base = wid * (NB // NW)
    pltpu.sync_copy(pbins_hbm.at[0, pl.ds(base, NB // NW), :], a_v)
    pltpu.sync_copy(pbins_hbm.at[1, pl.ds(base, NB // NW), :], b_v)

    @pl.loop(0, NB // NW)
    def _(i):
        for j in range(H // 16):
            sl = (i, pl.ds(j * 16, 16))
            a_v[sl] = a_v[sl] + b_v[sl]

    pltpu.sync_copy(a_v, bins_hbm.at[pl.ds(base, NB // NW), :])


def _sc_gather_kernel(bins_hbm, xs_hbm, ys_hbm, out_hbm,
                      rows_v, idx_v, x_v, y_v):
    c = lax.axis_index("c")
    s = lax.axis_index("s")
    wid = c * 16 + s

    @pl.loop(0, ITERS)
    def _(it):
        cid = it * NW + wid

        @pl.when(cid < NCHUNK)
        def _():
            _sc_ids(xs_hbm, ys_hbm, x_v, y_v, idx_v, cid)
            pltpu.sync_copy(bins_hbm.at[idx_v.at[0]], rows_v)

        @pl.when(cid < NFULL)
        def _():
            pltpu.sync_copy(rows_v, out_hbm.at[pl.ds(cid * CHUNK, CHUNK), :])

        @pl.when(cid == NFULL)
        def _():
            pltpu.sync_copy(rows_v.at[pl.ds(0, TAIL), :],
                            out_hbm.at[pl.ds(NFULL * CHUNK, TAIL), :])


def kernel(coords, hidden_state, cell_state, W_ih, W_hh, b_ih, b_hh):
    xs = coords[:, 0]
    ys = coords[:, 1]
    whh = W_hh.T.astype(jnp.bfloat16)
    wf = jnp.concatenate(
        [W_ih.T, (b_ih + b_hh)[None, :]], axis=0).astype(jnp.bfloat16)
    feat = jnp.concatenate(
        [xs.reshape(NBLK, 1, R), ys.reshape(NBLK, 1, R),
         jnp.ones((NBLK, 1, R), jnp.float32)],
        axis=1).astype(jnp.bfloat16)

    c_new, h_new = pl.pallas_call(
        _lstm_kernel,
        grid=(NBLK,),
        in_specs=[
            pl.BlockSpec((R, H), lambda i: (i, 0)),
            pl.BlockSpec((R, H), lambda i: (i, 0)),
            pl.BlockSpec((1, 3, R), lambda i: (i, 0, 0)),
            pl.BlockSpec((H, 4 * H), lambda i: (0, 0)),
            pl.BlockSpec((3, 4 * H), lambda i: (0, 0)),
        ],
        out_specs=[
            pl.BlockSpec((R, H), lambda i: (i, 0)),
            pl.BlockSpec((R, H), lambda i: (i, 0)),
        ],
        out_shape=[
            jax.ShapeDtypeStruct((N, H), jnp.float32),
            jax.ShapeDtypeStruct((N, H), jnp.float32),
        ],
    )(hidden_state, cell_state, feat, whh, wf)

    zeros = jnp.zeros((64, H), jnp.float32)

    scatter = pl.kernel(
        _sc_scatter_kernel,
        out_type=jax.ShapeDtypeStruct((2, NB, H), jnp.float32),
        mesh=_mesh,
        scratch_types=[
            pltpu.VMEM((CHUNK, H), jnp.float32),
            pltpu.VMEM((1, CHUNK), jnp.int32),
            pltpu.VMEM((CHUNK,), jnp.float32),
            pltpu.VMEM((CHUNK,), jnp.float32),
            pltpu.VMEM_SHARED((NB1, H), jnp.float32),
        ],
    )
    pbins = scatter(h_new, xs, ys, zeros)

    merge = pl.kernel(
        _sc_merge_kernel,
        out_type=jax.ShapeDtypeStruct((NB1, H), jnp.float32),
        mesh=_mesh,
        scratch_types=[
            pltpu.VMEM((NB // NW, H), jnp.float32),
            pltpu.VMEM((NB // NW, H), jnp.float32),
        ],
    )
    bins = merge(pbins)

    gather = pl.kernel(
        _sc_gather_kernel,
        out_type=jax.ShapeDtypeStruct((N, H), jnp.float32),
        mesh=_mesh,
        scratch_types=[
            pltpu.VMEM((CHUNK, H), jnp.float32),
            pltpu.VMEM((1, CHUNK), jnp.int32),
            pltpu.VMEM((CHUNK,), jnp.float32),
            pltpu.VMEM((CHUNK,), jnp.float32),
        ],
    )
    h_social = gather(bins, xs, ys)

    return (h_social, c_new)


# speedup vs baseline: 1.3658x; 1.2229x over previous
import jax
import jax.numpy as jnp
from jax import lax
from jax.experimental import pallas as pl
from jax.experimental.pallas import tpu as pltpu
from jax.experimental.pallas import tpu_sc as plsc

N = 100000
H = 128
G = 32
NB = G * G
NB1 = NB + 1
R = 2000
NBLK = N // R

CHUNK = 128
NFULL = N // CHUNK
TAIL = N - NFULL * CHUNK
NCHUNK = NFULL + 1
NW = 32
CPW = (NCHUNK + NW - 1) // NW
SPAN = CPW * CHUNK
LAST_SPAN = N - (NW - 1) * SPAN

_mesh = plsc.VectorSubcoreMesh(core_axis_name="c", subcore_axis_name="s")


def _tsig(x):
    return 0.5 * jnp.tanh(0.5 * x) + 0.5


def _lstm_kernel(hid_ref, cell_ref, feat_ref, whh_ref, wf_ref,
                 cnew_ref, hnew_ref):
    hid16 = hid_ref[...].astype(jnp.bfloat16)
    feat_blk = feat_ref[0]
    gates = (jnp.dot(hid16, whh_ref[...], preferred_element_type=jnp.float32)
             + lax.dot_general(feat_blk, wf_ref[...],
                               (((0,), (0,)), ((), ())),
                               preferred_element_type=jnp.float32))
    ii = _tsig(gates[:, :H])
    ff = _tsig(gates[:, H:2 * H])
    gg = jnp.tanh(gates[:, 2 * H:3 * H])
    oo = _tsig(gates[:, 3 * H:])
    c_new = ff * cell_ref[...] + ii * gg
    cnew_ref[...] = c_new
    hnew_ref[...] = oo * jnp.tanh(c_new)


def _stage_coords_and_ids(xs_hbm, ys_hbm, xa_v, ya_v, idx_v, wid):
    base = wid * SPAN

    @pl.when(wid < NW - 1)
    def _():
        pltpu.sync_copy(xs_hbm.at[pl.ds(base, SPAN)], xa_v)
        pltpu.sync_copy(ys_hbm.at[pl.ds(base, SPAN)], ya_v)

    @pl.when(wid == NW - 1)
    def _():
        pltpu.sync_copy(xs_hbm.at[pl.ds(base, LAST_SPAN)],
                        xa_v.at[pl.ds(0, LAST_SPAN)])
        pltpu.sync_copy(ys_hbm.at[pl.ds(base, LAST_SPAN)],
                        ya_v.at[pl.ds(0, LAST_SPAN)])

    @pl.loop(0, CPW)
    def _(j):
        for k in range(CHUNK // 16):
            xs = xa_v[pl.ds(j * CHUNK + 16 * k, 16)]
            ys = ya_v[pl.ds(j * CHUNK + 16 * k, 16)]
            ix = jnp.clip((jnp.clip(xs, 0.0, 1.0) * G).astype(jnp.int32),
                          0, G - 1)
            iy = jnp.clip((jnp.clip(ys, 0.0, 1.0) * G).astype(jnp.int32),
                          0, G - 1)
            ids = ix * G + iy
            row = base + j * CHUNK + 16 * k + lax.iota(jnp.int32, 16)
            idx_v[j, pl.ds(16 * k, 16)] = jnp.where(row < N, ids, NB)


def _rows_copy(hnew_hbm, buf_v, wid, j, sem):
    cid = wid * CPW + j
    full = jnp.logical_and(j < CPW, cid < NFULL)
    tail = jnp.logical_and(j < CPW, cid == NFULL)
    cp_full = pltpu.make_async_copy(
        hnew_hbm.at[pl.ds(cid * CHUNK, CHUNK), :], buf_v, sem)
    cp_tail = pltpu.make_async_copy(
        hnew_hbm.at[pl.ds(NFULL * CHUNK, TAIL), :],
        buf_v.at[pl.ds(0, TAIL), :], sem)
    return full, tail, cp_full, cp_tail


def _sc_scatter_kernel(hnew_hbm, xs_hbm, ys_hbm, zeros_hbm, pbins_hbm,
                       rows0_v, rows1_v, idx_v, xa_v, ya_v, bins_sh,
                       sem0, sem1):
    c = lax.axis_index("c")
    s = lax.axis_index("s")
    wid = c * 16 + s
    pltpu.sync_copy(zeros_hbm, bins_sh.at[pl.ds(s * 64, 64), :])
    _stage_coords_and_ids(xs_hbm, ys_hbm, xa_v, ya_v, idx_v, wid)
    plsc.subcore_barrier()

    def start(j, buf, sem):
        full, tail, cp_full, cp_tail = _rows_copy(hnew_hbm, buf, wid, j, sem)
        pl.when(full)(cp_full.start)
        pl.when(tail)(cp_tail.start)

    def wait(j, buf, sem):
        full, tail, cp_full, cp_tail = _rows_copy(hnew_hbm, buf, wid, j, sem)
        pl.when(full)(cp_full.wait)
        pl.when(tail)(cp_tail.wait)

    def scatter(j, buf):
        cid = wid * CPW + j

        @pl.when(jnp.logical_and(j < CPW, cid < NCHUNK))
        def _():
            pltpu.sync_copy(buf, bins_sh.at[idx_v.at[j]], add=True)

    start(0, rows0_v, sem0)

    @pl.loop(0, (CPW + 1) // 2)
    def _(k):
        j0 = 2 * k
        j1 = 2 * k + 1
        wait(j0, rows0_v, sem0)
        start(j1, rows1_v, sem1)
        scatter(j0, rows0_v)
        wait(j1, rows1_v, sem1)
        start(j0 + 2, rows0_v, sem0)
        scatter(j1, rows1_v)

    plsc.subcore_barrier()
    pltpu.sync_copy(bins_sh.at[pl.ds(s * 64, 64), :],
                    pbins_hbm.at[c, pl.ds(s * 64, 64), :])


def _sc_merge_kernel(pbins_hbm, bins_hbm, a_v, b_v):
    c = lax.axis_index("c")
    s = lax.axis_index("s")
    wid = c * 16 + s
    base = wid * (NB // NW)
    pltpu.sync_copy(pbins_hbm.at[0, pl.ds(base, NB // NW), :], a_v)
    pltpu.sync_copy(pbins_hbm.at[1, pl.ds(base, NB // NW), :], b_v)

    @pl.loop(0, NB // NW)
    def _(i):
        for j in range(H // 16):
            sl = (i, pl.ds(j * 16, 16))
            a_v[sl] = a_v[sl] + b_v[sl]

    pltpu.sync_copy(a_v, bins_hbm.at[pl.ds(base, NB // NW), :])


def _sc_gather_kernel(bins_hbm, xs_hbm, ys_hbm, out_hbm,
                      rows0_v, rows1_v, idx_v, xa_v, ya_v, sem0, sem1):
    c = lax.axis_index("c")
    s = lax.axis_index("s")
    wid = c * 16 + s
    _stage_coords_and_ids(xs_hbm, ys_hbm, xa_v, ya_v, idx_v, wid)

    def valid(j):
        return jnp.logical_and(j < CPW, wid * CPW + j < NCHUNK)

    def start(j, buf, sem):
        cp = pltpu.make_async_copy(bins_hbm.at[idx_v.at[j]], buf, sem)
        pl.when(valid(j))(cp.start)

    def wait(j, buf, sem):
        cp = pltpu.make_async_copy(bins_hbm.at[idx_v.at[j]], buf, sem)
        pl.when(valid(j))(cp.wait)

    def write(j, buf):
        cid = wid * CPW + j

        @pl.when(jnp.logical_and(j < CPW, cid < NFULL))
        def _():
            pltpu.sync_copy(buf, out_hbm.at[pl.ds(cid * CHUNK, CHUNK), :])

        @pl.when(jnp.logical_and(j < CPW, cid == NFULL))
        def _():
            pltpu.sync_copy(buf.at[pl.ds(0, TAIL), :],
                            out_hbm.at[pl.ds(NFULL * CHUNK, TAIL), :])

    start(0, rows0_v, sem0)

    @pl.loop(0, (CPW + 1) // 2)
    def _(k):
        j0 = 2 * k
        j1 = 2 * k + 1
        wait(j0, rows0_v, sem0)
        start(j1, rows1_v, sem1)
        write(j0, rows0_v)
        wait(j1, rows1_v, sem1)
        start(j0 + 2, rows0_v, sem0)
        write(j1, rows1_v)


def kernel(coords, hidden_state, cell_state, W_ih, W_hh, b_ih, b_hh):
    xs = coords[:, 0]
    ys = coords[:, 1]
    whh = W_hh.T.astype(jnp.bfloat16)
    wf = jnp.concatenate(
        [W_ih.T, (b_ih + b_hh)[None, :]], axis=0).astype(jnp.bfloat16)
    feat = jnp.concatenate(
        [xs.reshape(NBLK, 1, R), ys.reshape(NBLK, 1, R),
         jnp.ones((NBLK, 1, R), jnp.float32)],
        axis=1).astype(jnp.bfloat16)

    c_new, h_new = pl.pallas_call(
        _lstm_kernel,
        grid=(NBLK,),
        in_specs=[
            pl.BlockSpec((R, H), lambda i: (i, 0)),
            pl.BlockSpec((R, H), lambda i: (i, 0)),
            pl.BlockSpec((1, 3, R), lambda i: (i, 0, 0)),
            pl.BlockSpec((H, 4 * H), lambda i: (0, 0)),
            pl.BlockSpec((3, 4 * H), lambda i: (0, 0)),
        ],
        out_specs=[
            pl.BlockSpec((R, H), lambda i: (i, 0)),
            pl.BlockSpec((R, H), lambda i: (i, 0)),
        ],
        out_shape=[
            jax.ShapeDtypeStruct((N, H), jnp.float32),
            jax.ShapeDtypeStruct((N, H), jnp.float32),
        ],
    )(hidden_state, cell_state, feat, whh, wf)

    zeros = jnp.zeros((64, H), jnp.float32)

    scatter = pl.kernel(
        _sc_scatter_kernel,
        out_type=jax.ShapeDtypeStruct((2, NB, H), jnp.float32),
        mesh=_mesh,
        scratch_types=[
            pltpu.VMEM((CHUNK, H), jnp.float32),
            pltpu.VMEM((CHUNK, H), jnp.float32),
            pltpu.VMEM((CPW, CHUNK), jnp.int32),
            pltpu.VMEM((SPAN,), jnp.float32),
            pltpu.VMEM((SPAN,), jnp.float32),
            pltpu.VMEM_SHARED((NB1, H), jnp.float32),
            pltpu.SemaphoreType.DMA,
            pltpu.SemaphoreType.DMA,
        ],
    )
    pbins = scatter(h_new, xs, ys, zeros)

    merge = pl.kernel(
        _sc_merge_kernel,
        out_type=jax.ShapeDtypeStruct((NB1, H), jnp.float32),
        mesh=_mesh,
        scratch_types=[
            pltpu.VMEM((NB // NW, H), jnp.float32),
            pltpu.VMEM((NB // NW, H), jnp.float32),
        ],
    )
    bins = merge(pbins)

    gather = pl.kernel(
        _sc_gather_kernel,
        out_type=jax.ShapeDtypeStruct((N, H), jnp.float32),
        mesh=_mesh,
        scratch_types=[
            pltpu.VMEM((CHUNK, H), jnp.float32),
            pltpu.VMEM((CHUNK, H), jnp.float32),
            pltpu.VMEM((CPW, CHUNK), jnp.int32),
            pltpu.VMEM((SPAN,), jnp.float32),
            pltpu.VMEM((SPAN,), jnp.float32),
            pltpu.SemaphoreType.DMA,
            pltpu.SemaphoreType.DMA,
        ],
    )
    h_social = gather(bins, xs, ys)

    return (h_social, c_new)


# speedup vs baseline: 1.5710x; 1.1503x over previous
import jax
import jax.numpy as jnp
from jax import lax
from jax.experimental import pallas as pl
from jax.experimental.pallas import tpu as pltpu
from jax.experimental.pallas import tpu_sc as plsc

N = 100000
H = 128
G = 32
NB = G * G
NB1 = NB + 1
R = 2000
NBLK = N // R

CHUNK = 128
NFULL = N // CHUNK
TAIL = N - NFULL * CHUNK
NCHUNK = NFULL + 1
NW = 32
CPW = (NCHUNK + NW - 1) // NW
SPAN = CPW * CHUNK
LAST_SPAN = N - (NW - 1) * SPAN

_mesh = plsc.VectorSubcoreMesh(core_axis_name="c", subcore_axis_name="s")


def _tsig(x):
    return 0.5 * jnp.tanh(0.5 * x) + 0.5


def _lstm_kernel(hid_ref, cell_ref, feat_ref, whh_ref, wf_ref,
                 cnew_ref, hnew_ref):
    hid16 = hid_ref[...].astype(jnp.bfloat16)
    feat_blk = feat_ref[0]
    gates = (jnp.dot(hid16, whh_ref[...], preferred_element_type=jnp.float32)
             + lax.dot_general(feat_blk, wf_ref[...],
                               (((0,), (0,)), ((), ())),
                               preferred_element_type=jnp.float32))
    ii = _tsig(gates[:, :H])
    ff = _tsig(gates[:, H:2 * H])
    gg = jnp.tanh(gates[:, 2 * H:3 * H])
    oo = _tsig(gates[:, 3 * H:])
    c_new = ff * cell_ref[...] + ii * gg
    cnew_ref[...] = c_new
    hnew_ref[...] = oo * jnp.tanh(c_new)


def _stage_coords_and_ids(xs_hbm, ys_hbm, xa_v, ya_v, idx_v, wid):
    base = wid * SPAN

    @pl.when(wid < NW - 1)
    def _():
        pltpu.sync_copy(xs_hbm.at[pl.ds(base, SPAN)], xa_v)
        pltpu.sync_copy(ys_hbm.at[pl.ds(base, SPAN)], ya_v)

    @pl.when(wid == NW - 1)
    def _():
        pltpu.sync_copy(xs_hbm.at[pl.ds(base, LAST_SPAN)],
                        xa_v.at[pl.ds(0, LAST_SPAN)])
        pltpu.sync_copy(ys_hbm.at[pl.ds(base, LAST_SPAN)],
                        ya_v.at[pl.ds(0, LAST_SPAN)])

    @pl.loop(0, CPW)
    def _(j):
        for k in range(CHUNK // 16):
            xs = xa_v[pl.ds(j * CHUNK + 16 * k, 16)]
            ys = ya_v[pl.ds(j * CHUNK + 16 * k, 16)]
            ix = jnp.clip((jnp.clip(xs, 0.0, 1.0) * G).astype(jnp.int32),
                          0, G - 1)
            iy = jnp.clip((jnp.clip(ys, 0.0, 1.0) * G).astype(jnp.int32),
                          0, G - 1)
            ids = ix * G + iy
            row = base + j * CHUNK + 16 * k + lax.iota(jnp.int32, 16)
            idx_v[j, pl.ds(16 * k, 16)] = jnp.where(row < N, ids, NB)


def _rows_copy(hnew_hbm, buf_v, wid, j, sem):
    cid = wid * CPW + j
    full = jnp.logical_and(j < CPW, cid < NFULL)
    tail = jnp.logical_and(j < CPW, cid == NFULL)
    cp_full = pltpu.make_async_copy(
        hnew_hbm.at[pl.ds(cid * CHUNK, CHUNK), :], buf_v, sem)
    cp_tail = pltpu.make_async_copy(
        hnew_hbm.at[pl.ds(NFULL * CHUNK, TAIL), :],
        buf_v.at[pl.ds(0, TAIL), :], sem)
    return full, tail, cp_full, cp_tail


def _sc_scatter_kernel(hnew_hbm, xs_hbm, ys_hbm, zeros_hbm, pbins_hbm,
                       rows0_v, rows1_v, idx_v, xa_v, ya_v, bins_sh,
                       sem0, sem1):
    c = lax.axis_index("c")
    s = lax.axis_index("s")
    wid = c * 16 + s
    pltpu.sync_copy(zeros_hbm, bins_sh.at[pl.ds(s * 64, 64), :])
    _stage_coords_and_ids(xs_hbm, ys_hbm, xa_v, ya_v, idx_v, wid)
    plsc.subcore_barrier()

    def start(j, buf, sem):
        full, tail, cp_full, cp_tail = _rows_copy(hnew_hbm, buf, wid, j, sem)
        pl.when(full)(cp_full.start)
        pl.when(tail)(cp_tail.start)

    def wait(j, buf, sem):
        full, tail, cp_full, cp_tail = _rows_copy(hnew_hbm, buf, wid, j, sem)
        pl.when(full)(cp_full.wait)
        pl.when(tail)(cp_tail.wait)

    def scatter(j, buf):
        cid = wid * CPW + j

        @pl.when(jnp.logical_and(j < CPW, cid < NCHUNK))
        def _():
            pltpu.sync_copy(buf, bins_sh.at[idx_v.at[j]], add=True)

    start(0, rows0_v, sem0)

    @pl.loop(0, (CPW + 1) // 2)
    def _(k):
        j0 = 2 * k
        j1 = 2 * k + 1
        wait(j0, rows0_v, sem0)
        start(j1, rows1_v, sem1)
        scatter(j0, rows0_v)
        wait(j1, rows1_v, sem1)
        start(j0 + 2, rows0_v, sem0)
        scatter(j1, rows1_v)

    plsc.subcore_barrier()
    pltpu.sync_copy(bins_sh.at[pl.ds(s * 64, 64), :],
                    pbins_hbm.at[c, pl.ds(s * 64, 64), :])


def _sc_gather_kernel(pbins_hbm, xs_hbm, ys_hbm, out_hbm,
                      rows0_v, rows1_v, idx_v, xa_v, ya_v, a_v, b_v, bins_sh,
                      semg0, semg1, semw0, semw1):
    c = lax.axis_index("c")
    s = lax.axis_index("s")
    wid = c * 16 + s
    mb = s * (NB // 16)
    pltpu.sync_copy(pbins_hbm.at[0, pl.ds(mb, NB // 16), :], a_v)
    pltpu.sync_copy(pbins_hbm.at[1, pl.ds(mb, NB // 16), :], b_v)

    @pl.loop(0, NB // 16)
    def _(i):
        for j in range(H // 16):
            sl = (i, pl.ds(j * 16, 16))
            a_v[sl] = a_v[sl] + b_v[sl]

    pltpu.sync_copy(a_v, bins_sh.at[pl.ds(mb, NB // 16), :])
    _stage_coords_and_ids(xs_hbm, ys_hbm, xa_v, ya_v, idx_v, wid)
    plsc.subcore_barrier()

    def valid(j):
        return jnp.logical_and(j < CPW, wid * CPW + j < NCHUNK)

    def start_g(j, buf, sem):
        cp = pltpu.make_async_copy(bins_sh.at[idx_v.at[j]], buf, sem)
        pl.when(valid(j))(cp.start)

    def wait_g(j, buf, sem):
        cp = pltpu.make_async_copy(bins_sh.at[idx_v.at[j]], buf, sem)
        pl.when(valid(j))(cp.wait)

    def _write_copies(j, buf, sem):
        cid = wid * CPW + j
        full = jnp.logical_and(j < CPW, cid < NFULL)
        tail = jnp.logical_and(j < CPW, cid == NFULL)
        cp_full = pltpu.make_async_copy(
            buf, out_hbm.at[pl.ds(cid * CHUNK, CHUNK), :], sem)
        cp_tail = pltpu.make_async_copy(
            buf.at[pl.ds(0, TAIL), :],
            out_hbm.at[pl.ds(NFULL * CHUNK, TAIL), :], sem)
        return full, tail, cp_full, cp_tail

    def start_w(j, buf, sem):
        full, tail, cp_full, cp_tail = _write_copies(j, buf, sem)
        pl.when(full)(cp_full.start)
        pl.when(tail)(cp_tail.start)

    def wait_w(j, buf, sem):
        full, tail, cp_full, cp_tail = _write_copies(j, buf, sem)
        pl.when(full)(cp_full.wait)
        pl.when(tail)(cp_tail.wait)

    start_g(0, rows0_v, semg0)
    start_g(1, rows1_v, semg1)

    @pl.loop(0, (CPW + 1) // 2)
    def _(k):
        j0 = 2 * k
        j1 = 2 * k + 1
        wait_g(j0, rows0_v, semg0)
        start_w(j0, rows0_v, semw0)
        wait_g(j1, rows1_v, semg1)
        start_w(j1, rows1_v, semw1)
        wait_w(j0, rows0_v, semw0)
        start_g(j0 + 2, rows0_v, semg0)
        wait_w(j1, rows1_v, semw1)
        start_g(j1 + 2, rows1_v, semg1)


def kernel(coords, hidden_state, cell_state, W_ih, W_hh, b_ih, b_hh):
    xs = coords[:, 0]
    ys = coords[:, 1]
    whh = W_hh.T.astype(jnp.bfloat16)
    wf = jnp.concatenate(
        [W_ih.T, (b_ih + b_hh)[None, :]], axis=0).astype(jnp.bfloat16)
    feat = jnp.concatenate(
        [xs.reshape(NBLK, 1, R), ys.reshape(NBLK, 1, R),
         jnp.ones((NBLK, 1, R), jnp.float32)],
        axis=1).astype(jnp.bfloat16)

    c_new, h_new = pl.pallas_call(
        _lstm_kernel,
        grid=(NBLK,),
        in_specs=[
            pl.BlockSpec((R, H), lambda i: (i, 0)),
            pl.BlockSpec((R, H), lambda i: (i, 0)),
            pl.BlockSpec((1, 3, R), lambda i: (i, 0, 0)),
            pl.BlockSpec((H, 4 * H), lambda i: (0, 0)),
            pl.BlockSpec((3, 4 * H), lambda i: (0, 0)),
        ],
        out_specs=[
            pl.BlockSpec((R, H), lambda i: (i, 0)),
            pl.BlockSpec((R, H), lambda i: (i, 0)),
        ],
        out_shape=[
            jax.ShapeDtypeStruct((N, H), jnp.float32),
            jax.ShapeDtypeStruct((N, H), jnp.float32),
        ],
    )(hidden_state, cell_state, feat, whh, wf)

    zeros = jnp.zeros((64, H), jnp.float32)

    scatter = pl.kernel(
        _sc_scatter_kernel,
        out_type=jax.ShapeDtypeStruct((2, NB, H), jnp.float32),
        mesh=_mesh,
        scratch_types=[
            pltpu.VMEM((CHUNK, H), jnp.float32),
            pltpu.VMEM((CHUNK, H), jnp.float32),
            pltpu.VMEM((CPW, CHUNK), jnp.int32),
            pltpu.VMEM((SPAN,), jnp.float32),
            pltpu.VMEM((SPAN,), jnp.float32),
            pltpu.VMEM_SHARED((NB1, H), jnp.float32),
            pltpu.SemaphoreType.DMA,
            pltpu.SemaphoreType.DMA,
        ],
    )
    pbins = scatter(h_new, xs, ys, zeros)

    gather = pl.kernel(
        _sc_gather_kernel,
        out_type=jax.ShapeDtypeStruct((N, H), jnp.float32),
        mesh=_mesh,
        scratch_types=[
            pltpu.VMEM((CHUNK, H), jnp.float32),
            pltpu.VMEM((CHUNK, H), jnp.float32),
            pltpu.VMEM((CPW, CHUNK), jnp.int32),
            pltpu.VMEM((SPAN,), jnp.float32),
            pltpu.VMEM((SPAN,), jnp.float32),
            pltpu.VMEM((NB // 16, H), jnp.float32),
            pltpu.VMEM((NB // 16, H), jnp.float32),
            pltpu.VMEM_SHARED((NB1, H), jnp.float32),
            pltpu.SemaphoreType.DMA,
            pltpu.SemaphoreType.DMA,
            pltpu.SemaphoreType.DMA,
            pltpu.SemaphoreType.DMA,
        ],
    )
    h_social = gather(pbins, xs, ys)

    return (h_social, c_new)
